# baseline (speedup 1.0000x reference)
import functools
import math

import jax
import jax.numpy as jnp
import numpy as np
from jax import lax
from jax.experimental import pallas as pl
from jax.experimental.pallas import tpu as pltpu
from jax.experimental.pallas import tpu_sc as plsc

VOCAB = 100000
D_MODEL = 768
MAX_POS = 2048
_SCALE = math.sqrt(float(D_MODEL))
_LANES = 16
_CHUNK = 16
_DEPTH = 5


def _positional_encoding_np(length, depth):
    depth_h = depth / 2
    positions = np.arange(length)[:, np.newaxis]
    depths = np.arange(depth_h)[np.newaxis, :] / depth_h
    angle_rates = 1 / 10000 ** depths
    angle_rads = positions * angle_rates
    return np.concatenate(
        [np.sin(angle_rads), np.cos(angle_rads)], axis=-1
    ).astype(np.float32)


@functools.partial(jax.jit, static_argnums=(3, 4))
def _run(xr, pos, table, batch, seq_len):
    info = plsc.get_sparse_core_info()
    nc, ns = info.num_cores, info.num_subcores
    nw = nc * ns
    t_span = seq_len // nw
    b_per_w = batch * t_span
    quarters = t_span // _CHUNK
    n_chunks = batch * quarters
    cols16 = D_MODEL // _LANES
    n_rows = batch * seq_len

    mesh = plsc.VectorSubcoreMesh(core_axis_name="c", subcore_axis_name="s")

    @functools.partial(
        pl.kernel,
        mesh=mesh,
        out_type=jax.ShapeDtypeStruct((n_rows, D_MODEL), jnp.float32),
        scratch_types=[
            pltpu.VMEM((b_per_w,), jnp.int32),
        ]
        + [pltpu.VMEM((_CHUNK, D_MODEL), jnp.float32)] * (2 * _DEPTH)
        + [pltpu.SemaphoreType.DMA] * (3 * _DEPTH),
    )
    def body(x_hbm, pos_hbm, table_hbm, out_hbm,
             idx_v,
             g0, g1, g2, g3, g4, p0, p1, p2, p3, p4,
             gs0, gs1, gs2, gs3, gs4,
             ps0, ps1, ps2, ps3, ps4,
             os0, os1, os2, os3, os4):
        g = (g0, g1, g2, g3, g4)
        p = (p0, p1, p2, p3, p4)
        gsem = (gs0, gs1, gs2, gs3, gs4)
        psem = (ps0, ps1, ps2, ps3, ps4)
        osem = (os0, os1, os2, os3, os4)
        wid = lax.axis_index("s") * nc + lax.axis_index("c")
        t0 = wid * t_span
        pltpu.sync_copy(x_hbm.at[pl.ds(wid * b_per_w, b_per_w)], idx_v)

        def seg(j):
            return j % batch, j // batch

        def start(j):
            b, q = seg(j)
            buf = j % _DEPTH
            gh = pltpu.async_copy(
                table_hbm.at[
                    idx_v.at[pl.ds(b * t_span + q * _CHUNK, _CHUNK)]],
                g[buf], gsem[buf])
            lh = pltpu.async_copy(
                pos_hbm.at[pl.ds(t0 + q * _CHUNK, _CHUNK)], p[buf],
                psem[buf])
            return gh, lh

        gh = [None] * _DEPTH
        sh = [None] * _DEPTH
        for j in range(min(_DEPTH - 1, n_chunks)):
            gh[j % _DEPTH] = start(j)
        for j in range(n_chunks):
            buf = j % _DEPTH
            b, q = seg(j)
            if j + _DEPTH - 1 < n_chunks:
                nb = (j + _DEPTH - 1) % _DEPTH
                if sh[nb] is not None:
                    sh[nb].wait()
                    sh[nb] = None
                gh[nb] = start(j + _DEPTH - 1)
            gh[buf][0].wait()
            gh[buf][1].wait()

            @plsc.parallel_loop(0, _CHUNK, 1, unroll=1)
            def _(r):
                for c in range(cols16):
                    sl = pl.ds(c * _LANES, _LANES)
                    plsc.addupdate(p[buf].at[r, sl], g[buf][r, sl] * _SCALE)

            sh[buf] = pltpu.async_copy(
                p[buf],
                out_hbm.at[pl.ds(b * seq_len + t0 + q * _CHUNK, _CHUNK)],
                osem[buf])
        for h in sh:
            if h is not None:
                h.wait()

    return body(xr, pos, table)


def kernel(x, table):
    b, t = x.shape
    nw = 32
    t_span = t // nw
    xr = (x.reshape(b, nw, t_span).transpose(1, 0, 2).reshape(b * t)
          .astype(jnp.int32))
    pos = jnp.asarray(_positional_encoding_np(MAX_POS, D_MODEL))
    out = _run(xr, pos, table, b, t)
    return out.reshape(b, t, D_MODEL)


# speedup vs baseline: 1.1344x; 1.1344x over previous
import functools
import math

import jax
import jax.numpy as jnp
import numpy as np
from jax import lax
from jax.experimental import pallas as pl
from jax.experimental.pallas import tpu as pltpu
from jax.experimental.pallas import tpu_sc as plsc

VOCAB = 100000
D_MODEL = 768
MAX_POS = 2048
_SCALE = math.sqrt(float(D_MODEL))
_LANES = 16
_CHUNK = 16
_GDEPTH = 4
_PDEPTH = 2


def _positional_encoding_np(length, depth):
    depth_h = depth / 2
    positions = np.arange(length)[:, np.newaxis]
    depths = np.arange(depth_h)[np.newaxis, :] / depth_h
    angle_rates = 1 / 10000 ** depths
    angle_rads = positions * angle_rates
    return np.concatenate(
        [np.sin(angle_rads), np.cos(angle_rads)], axis=-1
    ).astype(np.float32)


@functools.partial(jax.jit, static_argnums=(3, 4))
def _run(xr, pos, table, batch, seq_len):
    info = plsc.get_sparse_core_info()
    nc, ns = info.num_cores, info.num_subcores
    nw = nc * ns
    t_span = seq_len // nw
    b_per_w = batch * t_span
    quarters = t_span // _CHUNK
    n_chunks = batch * quarters
    cols16 = D_MODEL // _LANES
    n_rows = batch * seq_len

    mesh = plsc.VectorSubcoreMesh(core_axis_name="c", subcore_axis_name="s")

    @functools.partial(
        pl.kernel,
        mesh=mesh,
        out_type=jax.ShapeDtypeStruct((n_rows, D_MODEL), jnp.float32),
        scratch_types=[
            pltpu.VMEM((b_per_w,), jnp.int32),
            pltpu.VMEM((t_span, D_MODEL), jnp.float32),
        ]
        + [pltpu.VMEM((_CHUNK, D_MODEL), jnp.float32)] * (_GDEPTH + _PDEPTH)
        + [pltpu.SemaphoreType.DMA] * (1 + _GDEPTH + _PDEPTH),
    )
    def body(x_hbm, pos_hbm, table_hbm, out_hbm,
             idx_v, pos_v,
             g0, g1, g2, g3, p0, p1,
             psem, gs0, gs1, gs2, gs3, os0, os1):
        g = (g0, g1, g2, g3)
        p = (p0, p1)
        gsem = (gs0, gs1, gs2, gs3)
        osem = (os0, os1)
        wid = lax.axis_index("s") * nc + lax.axis_index("c")
        t0 = wid * t_span
        pltpu.sync_copy(x_hbm.at[pl.ds(wid * b_per_w, b_per_w)], idx_v)

        def seg(j):
            return j % batch, j // batch

        def start_gather(j):
            b, q = seg(j)
            return pltpu.async_copy(
                table_hbm.at[
                    idx_v.at[pl.ds(b * t_span + q * _CHUNK, _CHUNK)]],
                g[j % _GDEPTH], gsem[j % _GDEPTH])

        ph = pltpu.async_copy(pos_hbm.at[pl.ds(t0, t_span)], pos_v, psem)
        gh = [None] * _GDEPTH
        sh = [None] * _PDEPTH
        for j in range(min(_GDEPTH - 1, n_chunks)):
            gh[j % _GDEPTH] = start_gather(j)
        ph.wait()
        for j in range(n_chunks):
            buf = j % _GDEPTH
            pbuf = j % _PDEPTH
            b, q = seg(j)
            if j + _GDEPTH - 1 < n_chunks:
                nb = (j + _GDEPTH - 1) % _GDEPTH
                gh[nb] = start_gather(j + _GDEPTH - 1)
            gh[buf].wait()
            if sh[pbuf] is not None:
                sh[pbuf].wait()
                sh[pbuf] = None
            pr0 = q * _CHUNK

            @plsc.parallel_loop(0, _CHUNK, 1, unroll=1)
            def _(r):
                for c in range(cols16):
                    sl = pl.ds(c * _LANES, _LANES)
                    p[pbuf][r, sl] = (
                        g[buf][r, sl] * _SCALE + pos_v[pr0 + r, sl])

            sh[pbuf] = pltpu.async_copy(
                p[pbuf],
                out_hbm.at[pl.ds(b * seq_len + t0 + q * _CHUNK, _CHUNK)],
                osem[pbuf])
        for h in sh:
            if h is not None:
                h.wait()

    return body(xr, pos, table)


def kernel(x, table):
    b, t = x.shape
    nw = 32
    t_span = t // nw
    xr = (x.reshape(b, nw, t_span).transpose(1, 0, 2).reshape(b * t)
          .astype(jnp.int32))
    pos = jnp.asarray(_positional_encoding_np(MAX_POS, D_MODEL))
    out = _run(xr, pos, table, b, t)
    return out.reshape(b, t, D_MODEL)


# speedup vs baseline: 1.1464x; 1.0106x over previous
import functools
import math

import jax
import jax.numpy as jnp
import numpy as np
from jax import lax
from jax.experimental import pallas as pl
from jax.experimental.pallas import tpu as pltpu
from jax.experimental.pallas import tpu_sc as plsc

VOCAB = 100000
D_MODEL = 768
MAX_POS = 2048
_SCALE = math.sqrt(float(D_MODEL))
_LANES = 16
_CHUNK = 16
_GDEPTH = 4
_PDEPTH = 2


def _positional_encoding_np(length, depth):
    depth_h = depth / 2
    positions = np.arange(length)[:, np.newaxis]
    depths = np.arange(depth_h)[np.newaxis, :] / depth_h
    angle_rates = 1 / 10000 ** depths
    angle_rads = positions * angle_rates
    return np.concatenate(
        [np.sin(angle_rads), np.cos(angle_rads)], axis=-1
    ).astype(np.float32)


@functools.partial(jax.jit, static_argnums=(3, 4))
def _run(xr, pos, table, batch, seq_len):
    info = plsc.get_sparse_core_info()
    nc, ns = info.num_cores, info.num_subcores
    nw = nc * ns
    t_span = seq_len // nw
    b_per_w = batch * t_span
    quarters = t_span // _CHUNK
    n_chunks = batch * quarters
    cols16 = D_MODEL // _LANES
    n_rows = batch * seq_len

    mesh = plsc.VectorSubcoreMesh(core_axis_name="c", subcore_axis_name="s")

    @functools.partial(
        pl.kernel,
        mesh=mesh,
        out_type=jax.ShapeDtypeStruct((n_rows, D_MODEL), jnp.float32),
        scratch_types=[
            pltpu.VMEM((b_per_w,), jnp.int32),
            pltpu.VMEM((t_span, D_MODEL), jnp.float32),
        ]
        + [pltpu.VMEM((_CHUNK, D_MODEL), jnp.float32)] * (_GDEPTH + _PDEPTH)
        + [pltpu.SemaphoreType.DMA] * (1 + _GDEPTH + _PDEPTH),
    )
    def body(x_hbm, pos_hbm, table_hbm, out_hbm,
             idx_v, pos_v,
             g0, g1, g2, g3, p0, p1,
             psem, gs0, gs1, gs2, gs3, os0, os1):
        g = (g0, g1, g2, g3)
        p = (p0, p1)
        gsem = (gs0, gs1, gs2, gs3)
        osem = (os0, os1)
        wid = lax.axis_index("s") * nc + lax.axis_index("c")
        t0 = wid * t_span
        ph = pltpu.async_copy(pos_hbm.at[pl.ds(t0, t_span)], pos_v, psem)
        pltpu.sync_copy(x_hbm.at[pl.ds(wid * b_per_w, b_per_w)], idx_v)

        def seg(j):
            return j % batch, j // batch

        def start_gather(j):
            b, q = seg(j)
            return pltpu.async_copy(
                table_hbm.at[
                    idx_v.at[pl.ds(b * t_span + q * _CHUNK, _CHUNK)]],
                g[j % _GDEPTH], gsem[j % _GDEPTH])

        gh = [None] * _GDEPTH
        sh = [None] * _PDEPTH
        for j in range(min(_GDEPTH - 1, n_chunks)):
            gh[j % _GDEPTH] = start_gather(j)
        ph.wait()
        for j in range(n_chunks):
            buf = j % _GDEPTH
            pbuf = j % _PDEPTH
            b, q = seg(j)
            if j + _GDEPTH - 1 < n_chunks:
                nb = (j + _GDEPTH - 1) % _GDEPTH
                gh[nb] = start_gather(j + _GDEPTH - 1)
            gh[buf].wait()
            if sh[pbuf] is not None:
                sh[pbuf].wait()
                sh[pbuf] = None
            pr0 = q * _CHUNK

            @plsc.parallel_loop(0, _CHUNK, 1, unroll=1)
            def _(r):
                for c in range(cols16):
                    sl = pl.ds(c * _LANES, _LANES)
                    p[pbuf][r, sl] = (
                        g[buf][r, sl] * _SCALE + pos_v[pr0 + r, sl])

            sh[pbuf] = pltpu.async_copy(
                p[pbuf],
                out_hbm.at[pl.ds(b * seq_len + t0 + q * _CHUNK, _CHUNK)],
                osem[pbuf])
        for h in sh:
            if h is not None:
                h.wait()

    return body(xr, pos, table)


def kernel(x, table):
    b, t = x.shape
    nw = 32
    t_span = t // nw
    xr = (x.reshape(b, nw, t_span).transpose(1, 0, 2).reshape(b * t)
          .astype(jnp.int32))
    pos = jnp.asarray(_positional_encoding_np(MAX_POS, D_MODEL))
    out = _run(xr, pos, table, b, t)
    return out.reshape(b, t, D_MODEL)
